# baseline (speedup 1.0000x reference)
import jax
import jax.numpy as jnp
import numpy as np
from jax.experimental import pallas as pl
from jax.experimental.pallas import tpu as pltpu

_N_BINS = 15
_LANES = 128

_STEP = np.float32(1.0 / _N_BINS)


def _bin_bounds():
    j = jax.lax.broadcasted_iota(jnp.int32, (1, _LANES), 1)
    jf = j.astype(jnp.float32)
    low = jnp.where(j < _N_BINS, jf * _STEP, 2.0)
    upp = jnp.where(j < _N_BINS, (jf + 1.0) * _STEP, 3.0)
    return low, upp


def _ece_body(n_total, p_ref, lab_ref, out_ref, acc_ref):
    i = pl.program_id(0)

    @pl.when(i == 0)
    def _init():
        acc_ref[...] = jnp.zeros_like(acc_ref)

    p = p_ref[...]
    lab = lab_ref[0, 0, :]
    r, c = p.shape

    rowmax = jnp.max(p, axis=1, keepdims=True)
    col = jax.lax.broadcasted_iota(jnp.int32, (r, c), 1)
    amin = jnp.min(jnp.where(p == rowmax, col, c), axis=1)
    accv = (amin == lab).astype(jnp.float32)[:, None]

    low, upp = _bin_bounds()
    mask = ((rowmax > low) & (rowmax <= upp)).astype(jnp.float32)
    acc_ref[0:1, :] += jnp.sum(mask, axis=0, keepdims=True)
    acc_ref[1:2, :] += jnp.sum(mask * rowmax, axis=0, keepdims=True)
    acc_ref[2:3, :] += jnp.sum(mask * accv, axis=0, keepdims=True)

    @pl.when(i == pl.num_programs(0) - 1)
    def _fin():
        cnt = acc_ref[0:1, :]
        sconf = acc_ref[1:2, :]
        sacc = acc_ref[2:3, :]
        nonempty = cnt > 0
        safe = jnp.where(nonempty, cnt, 1.0)
        per_bin = jnp.where(
            nonempty,
            jnp.abs(sconf / safe - sacc / safe) * (cnt * (1.0 / n_total)),
            0.0,
        )
        out_ref[...] = jnp.sum(per_bin, keepdims=True)


def kernel(probabilities, labels):
    n, c = probabilities.shape
    rows = 8000
    grid = n // rows
    labs = labels.astype(jnp.int32).reshape(grid, 1, rows)

    out = pl.pallas_call(
        lambda *refs: _ece_body(n, *refs),
        grid=(grid,),
        in_specs=[
            pl.BlockSpec((rows, c), lambda i: (i, 0)),
            pl.BlockSpec((1, 1, rows), lambda i: (i, 0, 0)),
        ],
        out_specs=pl.BlockSpec((1, 1), lambda i: (0, 0)),
        out_shape=jax.ShapeDtypeStruct((1, 1), jnp.float32),
        scratch_shapes=[pltpu.VMEM((8, _LANES), jnp.float32)],
    )(probabilities, labs)
    return out.reshape(1)


# speedup vs baseline: 1.5586x; 1.5586x over previous
import jax
import jax.numpy as jnp
import numpy as np
from jax.experimental import pallas as pl
from jax.experimental.pallas import tpu as pltpu

_N_BINS = 15
_LANES = 128
_STEP = np.float32(1.0 / _N_BINS)
_TILE = 128


def _ece_body(n_total, p_ref, lab_ref, out_ref, acc_ref):
    i = pl.program_id(0)

    @pl.when(i == 0)
    def _init():
        acc_ref[...] = jnp.zeros_like(acc_ref)

    r, c = p_ref.shape
    lab2d = lab_ref[0]

    s = jax.lax.broadcasted_iota(jnp.int32, (16, 1), 0)
    sf = s.astype(jnp.float32)
    low = sf * _STEP
    upp = (sf + 1.0) * _STEP

    row_iota = jax.lax.broadcasted_iota(jnp.int32, (c, _TILE), 0)

    def tile_sums(lo, w):
        tile = p_ref[pl.ds(lo, w), :]
        tt = jnp.swapaxes(tile, 0, 1)
        conf = jnp.max(tt, axis=0, keepdims=True)
        lab_t = lab2d[:, lo:lo + w]
        rowsel = row_iota[:, :w] == lab_t
        plab = jnp.max(jnp.where(rowsel, tt, -1.0), axis=0, keepdims=True)
        accv = (plab == conf).astype(jnp.float32)
        mask = ((conf > low) & (conf <= upp)).astype(jnp.float32)
        return mask, mask * conf, mask * accv

    n_full = r // _TILE
    tail = r - n_full * _TILE
    cnt_a = jnp.zeros((16, _LANES), jnp.float32)
    sconf_a = jnp.zeros((16, _LANES), jnp.float32)
    sacc_a = jnp.zeros((16, _LANES), jnp.float32)
    for t in range(n_full):
        m, mc, ma = tile_sums(t * _TILE, _TILE)
        cnt_a += m
        sconf_a += mc
        sacc_a += ma
    acc_ref[0:16, :] += cnt_a
    acc_ref[16:32, :] += sconf_a
    acc_ref[32:48, :] += sacc_a
    if tail:
        m, mc, ma = tile_sums(n_full * _TILE, tail)
        acc_ref[0:16, 0:tail] += m
        acc_ref[16:32, 0:tail] += mc
        acc_ref[32:48, 0:tail] += ma

    @pl.when(i == pl.num_programs(0) - 1)
    def _fin():
        cnt = jnp.sum(acc_ref[0:16, :], axis=1, keepdims=True)
        sconf = jnp.sum(acc_ref[16:32, :], axis=1, keepdims=True)
        sacc = jnp.sum(acc_ref[32:48, :], axis=1, keepdims=True)
        nonempty = cnt > 0
        safe = jnp.where(nonempty, cnt, 1.0)
        per_bin = jnp.where(
            nonempty,
            jnp.abs(sconf / safe - sacc / safe) * (cnt * (1.0 / n_total)),
            0.0,
        )
        out_ref[...] = jnp.sum(per_bin, axis=0, keepdims=True)


def kernel(probabilities, labels):
    n, c = probabilities.shape
    rows = 4000
    grid = n // rows
    labs = labels.astype(jnp.int32).reshape(grid, 1, rows)

    out = pl.pallas_call(
        lambda *refs: _ece_body(n, *refs),
        grid=(grid,),
        in_specs=[
            pl.BlockSpec((rows, c), lambda i: (i, 0)),
            pl.BlockSpec((1, 1, rows), lambda i: (i, 0, 0)),
        ],
        out_specs=pl.BlockSpec((1, 1), lambda i: (0, 0)),
        out_shape=jax.ShapeDtypeStruct((1, 1), jnp.float32),
        scratch_shapes=[pltpu.VMEM((48, _LANES), jnp.float32)],
    )(probabilities, labs)
    return out.reshape(1)
